# baseline (device time: 215537 ns/iter reference)
import functools

import jax
import jax.numpy as jnp
from jax import lax
from jax.experimental import pallas as pl
from jax.experimental.pallas import tpu as pltpu

N_DEV = 16
M_PER = 256
K_PER = 256
N_COLS = 2048

_RING = [0, 1, 5, 9, 13, 14, 10, 6, 2, 3, 7, 11, 15, 12, 8, 4]
_POS = [0] * N_DEV
for _p, _l in enumerate(_RING):
    _POS[_l] = _p


def _gelu(y):
    c = 0.7978845608028654
    return 0.5 * y * (1.0 + jnp.tanh(c * (y + 0.044715 * y * y * y)))


def kernel(x, w_mat):
    ring = jnp.array(_RING, dtype=jnp.int32)
    pos_of = jnp.array(_POS, dtype=jnp.int32)
    me = lax.axis_index("i").astype(jnp.int32)
    p = pos_of[me]
    prev_id = ring[(p - 1) % N_DEV]
    next_id = ring[(p + 1) % N_DEV]
    recv_blocks = ring[(p - 2 - jnp.arange(N_DEV - 1, dtype=jnp.int32)) % N_DEV]
    meta = jnp.concatenate([jnp.stack([prev_id, next_id]), recv_blocks])

    def body(x_ref, w_ref, meta_ref, out_ref, comm_ref, send_sems, recv_sems):
        prev_id = meta_ref[0]
        next_id = meta_ref[1]

        barrier = pltpu.get_barrier_semaphore()
        for nbr in (prev_id, next_id):
            pl.semaphore_signal(
                barrier, inc=1, device_id=(nbr,),
                device_id_type=pl.DeviceIdType.MESH,
            )
        pl.semaphore_wait(barrier, 2)

        b0 = prev_id
        comm_ref[0] = jnp.dot(
            x_ref[pl.ds(b0 * M_PER, M_PER), :], w_ref[:, :],
            preferred_element_type=jnp.float32,
        ).astype(jnp.bfloat16)

        for s in range(N_DEV - 1):
            send_slot = s % 2
            recv_slot = (s + 1) % 2
            rdma = pltpu.make_async_remote_copy(
                src_ref=comm_ref.at[send_slot],
                dst_ref=comm_ref.at[recv_slot],
                send_sem=send_sems.at[s],
                recv_sem=recv_sems.at[s],
                device_id=(next_id,),
                device_id_type=pl.DeviceIdType.MESH,
            )
            rdma.start()
            rdma.wait()

            b = meta_ref[2 + s]
            local = jnp.dot(
                x_ref[pl.ds(b * M_PER, M_PER), :], w_ref[:, :],
                preferred_element_type=jnp.float32,
            )
            if s < N_DEV - 2:
                comm_ref[recv_slot] = (
                    local + comm_ref[recv_slot].astype(jnp.float32)
                ).astype(jnp.bfloat16)
            else:
                out_ref[:, :] = _gelu(
                    local + comm_ref[recv_slot].astype(jnp.float32)
                )

    return pl.pallas_call(
        body,
        out_shape=jax.ShapeDtypeStruct((M_PER, N_COLS), jnp.float32),
        in_specs=[
            pl.BlockSpec(memory_space=pltpu.VMEM),
            pl.BlockSpec(memory_space=pltpu.VMEM),
            pl.BlockSpec(memory_space=pltpu.SMEM),
        ],
        out_specs=pl.BlockSpec(memory_space=pltpu.VMEM),
        scratch_shapes=[
            pltpu.VMEM((2, M_PER, N_COLS), jnp.bfloat16),
            pltpu.SemaphoreType.DMA((N_DEV - 1,)),
            pltpu.SemaphoreType.DMA((N_DEV - 1,)),
        ],
        compiler_params=pltpu.CompilerParams(collective_id=0),
    )(x, w_mat, meta)


# device time: 136488 ns/iter; 1.5792x vs baseline; 1.5792x over previous
import jax
import jax.numpy as jnp
from jax import lax
from jax.experimental import pallas as pl
from jax.experimental.pallas import tpu as pltpu

N_DEV = 16
M_PER = 256
N_COLS = 2048
N_HALF = N_COLS // 2

_RING = [0, 1, 5, 9, 13, 14, 10, 6, 2, 3, 7, 11, 15, 12, 8, 4]
_POS = [0] * N_DEV
for _p, _l in enumerate(_RING):
    _POS[_l] = _p


def _gelu(y):
    c = 0.7978845608028654
    return 0.5 * y * (1.0 + jnp.tanh(c * (y + 0.044715 * y * y * y)))


def kernel(x, w_mat):
    ring = jnp.array(_RING, dtype=jnp.int32)
    pos_of = jnp.array(_POS, dtype=jnp.int32)
    me = lax.axis_index("i").astype(jnp.int32)
    p = pos_of[me]
    prev_id = ring[(p - 1) % N_DEV]
    next_id = ring[(p + 1) % N_DEV]
    hops = jnp.arange(N_DEV - 1, dtype=jnp.int32)
    cw_blocks = ring[(p - 2 - hops) % N_DEV]
    ccw_blocks = ring[(p + 2 + hops) % N_DEV]
    meta = jnp.concatenate([jnp.stack([prev_id, next_id]), cw_blocks, ccw_blocks])

    def body(x_ref, w_ref, meta_ref, out_ref,
             cw_ref, ccw_ref, send_cw, recv_cw, send_ccw, recv_ccw):
        prev_id = meta_ref[0]
        next_id = meta_ref[1]

        barrier = pltpu.get_barrier_semaphore()
        for nbr in (prev_id, next_id):
            pl.semaphore_signal(
                barrier, inc=1, device_id=(nbr,),
                device_id_type=pl.DeviceIdType.MESH,
            )
        pl.semaphore_wait(barrier, 2)

        def partial(block, lo):
            return jnp.dot(
                x_ref[pl.ds(block * M_PER, M_PER), :],
                w_ref[:, lo:lo + N_HALF],
                preferred_element_type=jnp.float32,
            )

        cw_ref[0] = partial(prev_id, 0).astype(jnp.bfloat16)
        ccw_ref[0] = partial(next_id, N_HALF).astype(jnp.bfloat16)

        rdmas = []
        for s in range(N_DEV - 1):
            cw = pltpu.make_async_remote_copy(
                src_ref=cw_ref.at[s], dst_ref=cw_ref.at[s + 1],
                send_sem=send_cw.at[s], recv_sem=recv_cw.at[s],
                device_id=(next_id,), device_id_type=pl.DeviceIdType.MESH,
            )
            ccw = pltpu.make_async_remote_copy(
                src_ref=ccw_ref.at[s], dst_ref=ccw_ref.at[s + 1],
                send_sem=send_ccw.at[s], recv_sem=recv_ccw.at[s],
                device_id=(prev_id,), device_id_type=pl.DeviceIdType.MESH,
            )
            cw.start()
            ccw.start()
            rdmas.append((cw, ccw))

            local_cw = partial(meta_ref[2 + s], 0)
            local_ccw = partial(meta_ref[2 + (N_DEV - 1) + s], N_HALF)

            cw.wait_recv()
            if s < N_DEV - 2:
                cw_ref[s + 1] = (
                    local_cw + cw_ref[s + 1].astype(jnp.float32)
                ).astype(jnp.bfloat16)
            else:
                out_ref[:, 0:N_HALF] = _gelu(
                    local_cw + cw_ref[s + 1].astype(jnp.float32)
                )
            ccw.wait_recv()
            if s < N_DEV - 2:
                ccw_ref[s + 1] = (
                    local_ccw + ccw_ref[s + 1].astype(jnp.float32)
                ).astype(jnp.bfloat16)
            else:
                out_ref[:, N_HALF:N_COLS] = _gelu(
                    local_ccw + ccw_ref[s + 1].astype(jnp.float32)
                )

        for cw, ccw in rdmas:
            cw.wait_send()
            ccw.wait_send()

    return pl.pallas_call(
        body,
        out_shape=jax.ShapeDtypeStruct((M_PER, N_COLS), jnp.float32),
        in_specs=[
            pl.BlockSpec(memory_space=pltpu.VMEM),
            pl.BlockSpec(memory_space=pltpu.VMEM),
            pl.BlockSpec(memory_space=pltpu.SMEM),
        ],
        out_specs=pl.BlockSpec(memory_space=pltpu.VMEM),
        scratch_shapes=[
            pltpu.VMEM((N_DEV, M_PER, N_HALF), jnp.bfloat16),
            pltpu.VMEM((N_DEV, M_PER, N_HALF), jnp.bfloat16),
            pltpu.SemaphoreType.DMA((N_DEV - 1,)),
            pltpu.SemaphoreType.DMA((N_DEV - 1,)),
            pltpu.SemaphoreType.DMA((N_DEV - 1,)),
            pltpu.SemaphoreType.DMA((N_DEV - 1,)),
        ],
        compiler_params=pltpu.CompilerParams(collective_id=0),
    )(x, w_mat, meta)


# device time: 102212 ns/iter; 2.1087x vs baseline; 1.3353x over previous
import jax
import jax.numpy as jnp
from jax import lax
from jax.experimental import pallas as pl
from jax.experimental.pallas import tpu as pltpu

N_DEV = 16
M_PER = 256
N_COLS = 2048
N_HALF = N_COLS // 2
N_SUB = N_HALF // 2

_RING = [0, 1, 5, 9, 13, 14, 10, 6, 2, 3, 7, 11, 15, 12, 8, 4]
_POS = [0] * N_DEV
for _p, _l in enumerate(_RING):
    _POS[_l] = _p


def _gelu(y):
    c = 0.7978845608028654
    return 0.5 * y * (1.0 + jnp.tanh(c * (y + 0.044715 * y * y * y)))


def kernel(x, w_mat):
    ring = jnp.array(_RING, dtype=jnp.int32)
    pos_of = jnp.array(_POS, dtype=jnp.int32)
    me = lax.axis_index("i").astype(jnp.int32)
    p = pos_of[me]
    prev_id = ring[(p - 1) % N_DEV]
    next_id = ring[(p + 1) % N_DEV]
    hops = jnp.arange(N_DEV - 1, dtype=jnp.int32)
    cw_blocks = ring[(p - 2 - hops) % N_DEV]
    ccw_blocks = ring[(p + 2 + hops) % N_DEV]
    meta = jnp.concatenate([jnp.stack([prev_id, next_id]), cw_blocks, ccw_blocks])

    def body(x_ref, w_ref, meta_ref, out_ref,
             cw_ref, ccw_ref, send_cw, recv_cw, send_ccw, recv_ccw):
        prev_id = meta_ref[0]
        next_id = meta_ref[1]

        barrier = pltpu.get_barrier_semaphore()
        for nbr in (prev_id, next_id):
            pl.semaphore_signal(
                barrier, inc=1, device_id=(nbr,),
                device_id_type=pl.DeviceIdType.MESH,
            )
        pl.semaphore_wait(barrier, 2)

        def partial(block, lo, width):
            return jnp.dot(
                x_ref[pl.ds(block * M_PER, M_PER), :],
                w_ref[:, lo:lo + width],
                preferred_element_type=jnp.float32,
            )

        def make(comm, ssems, rsems, h, j, tgt):
            return pltpu.make_async_remote_copy(
                src_ref=comm.at[h, :, pl.ds(j * N_SUB, N_SUB)],
                dst_ref=comm.at[h + 1, :, pl.ds(j * N_SUB, N_SUB)],
                send_sem=ssems.at[h, j], recv_sem=rsems.at[h, j],
                device_id=(tgt,), device_id_type=pl.DeviceIdType.MESH,
            )

        cw_ref[0] = partial(prev_id, 0, N_HALF).astype(jnp.bfloat16)
        ccw_ref[0] = partial(next_id, N_HALF, N_HALF).astype(jnp.bfloat16)

        started = []

        def launch(comm, ssems, rsems, h, j, tgt):
            r = make(comm, ssems, rsems, h, j, tgt)
            r.start()
            started.append(r)
            return r

        inflight = []
        for j in range(2):
            a = launch(cw_ref, send_cw, recv_cw, 0, j, next_id)
            b = launch(ccw_ref, send_ccw, recv_ccw, 0, j, prev_id)
            inflight.append((a, b))

        for s in range(N_DEV - 1):
            last = s == N_DEV - 2
            local_cw = partial(meta_ref[2 + s], 0, N_HALF)
            local_ccw = partial(meta_ref[2 + (N_DEV - 1) + s], N_HALF, N_HALF)
            arriving, inflight = inflight, []

            for j in range(2):
                lo = j * N_SUB
                cw_in, ccw_in = arriving[j]
                cw_in.wait_recv()
                acc_cw = (
                    local_cw[:, lo:lo + N_SUB]
                    + cw_ref[s + 1, :, lo:lo + N_SUB].astype(jnp.float32)
                )
                if last:
                    out_ref[:, lo:lo + N_SUB] = _gelu(acc_cw)
                    nxt = None
                else:
                    cw_ref[s + 1, :, lo:lo + N_SUB] = acc_cw.astype(jnp.bfloat16)
                    nxt = launch(cw_ref, send_cw, recv_cw, s + 1, j, next_id)

                ccw_in.wait_recv()
                acc_ccw = (
                    local_ccw[:, lo:lo + N_SUB]
                    + ccw_ref[s + 1, :, lo:lo + N_SUB].astype(jnp.float32)
                )
                if last:
                    out_ref[:, N_HALF + lo:N_HALF + lo + N_SUB] = _gelu(acc_ccw)
                else:
                    ccw_ref[s + 1, :, lo:lo + N_SUB] = acc_ccw.astype(jnp.bfloat16)
                    nxt2 = launch(ccw_ref, send_ccw, recv_ccw, s + 1, j, prev_id)
                    inflight.append((nxt, nxt2))

        for r in started:
            r.wait_send()

    return pl.pallas_call(
        body,
        out_shape=jax.ShapeDtypeStruct((M_PER, N_COLS), jnp.float32),
        in_specs=[
            pl.BlockSpec(memory_space=pltpu.VMEM),
            pl.BlockSpec(memory_space=pltpu.VMEM),
            pl.BlockSpec(memory_space=pltpu.SMEM),
        ],
        out_specs=pl.BlockSpec(memory_space=pltpu.VMEM),
        scratch_shapes=[
            pltpu.VMEM((N_DEV, M_PER, N_HALF), jnp.bfloat16),
            pltpu.VMEM((N_DEV, M_PER, N_HALF), jnp.bfloat16),
            pltpu.SemaphoreType.DMA((N_DEV - 1, 2)),
            pltpu.SemaphoreType.DMA((N_DEV - 1, 2)),
            pltpu.SemaphoreType.DMA((N_DEV - 1, 2)),
            pltpu.SemaphoreType.DMA((N_DEV - 1, 2)),
        ],
        compiler_params=pltpu.CompilerParams(collective_id=0),
    )(x, w_mat, meta)
